# baseline (speedup 1.0000x reference)
import functools
import math

import jax
import jax.numpy as jnp
from jax import lax
from jax.experimental import pallas as pl
from jax.experimental.pallas import tpu as pltpu
from jax.experimental.pallas import tpu_sc as plsc

B, N, D, H = 2, 2048, 2048, 16
C = N // 2
DH = D // H
NC, NS, LANES = 2, 16, 16

_MESH = plsc.VectorSubcoreMesh(
    core_axis_name="c", subcore_axis_name="s", num_cores=NC, num_subcores=NS
)

_VPT = N // NS
_GPT = C // NS


def _scores_body(x_ref, w_ref, o_ref):
    xb = x_ref[...].astype(jnp.bfloat16).astype(jnp.float32)
    wb = w_ref[...].astype(jnp.bfloat16).astype(jnp.float32)
    o_ref[...] = jnp.sum(xb * wb, axis=1)[None, None, :]


def _scores(x2d, w_row):
    nblk = 8
    rows = (B * N) // nblk
    out = pl.pallas_call(
        _scores_body,
        grid=(nblk,),
        in_specs=[
            pl.BlockSpec((rows, D), lambda i: (i, 0)),
            pl.BlockSpec((1, D), lambda i: (0, 0)),
        ],
        out_specs=pl.BlockSpec((1, 1, rows), lambda i: (i, 0, 0)),
        out_shape=jax.ShapeDtypeStruct((nblk, 1, rows), jnp.float32),
    )(x2d, w_row)
    return out.reshape(B, N)


def _select_gather_body(
    scores_hbm, x_hbm,
    selfull_hbm, gate_hbm, xin_hbm,
    scores_v, posb, valb, gateb, selv, gatev, gidx, rowb, sem,
    sel_sh, gate_sh,
):
    b = lax.axis_index("c")
    s = lax.axis_index("s")

    pltpu.sync_copy(scores_hbm.at[pl.ds(pl.multiple_of(b * N, N), N)], scores_v)

    def dec(e):
        big = e >= jnp.uint32(0x80000000)
        bits = jnp.where(big, e & jnp.uint32(0x7FFFFFFF), ~e)
        return lax.bitcast_convert_type(bits, jnp.float32)

    def _lanesum(vec):
        tot = vec[0]
        for i in range(1, LANES):
            tot = tot + vec[i]
        return tot

    def count_cmp(thr, strict):
        def cb(j, acc):
            v = scores_v[pl.ds(j * LANES, LANES)]
            hit = (v > thr) if strict else (v >= thr)
            return acc + jnp.where(hit, 1, 0).astype(jnp.int32)

        acc = lax.fori_loop(
            0, N // LANES, cb, jnp.zeros((LANES,), jnp.int32)
        )
        return _lanesum(acc)

    def bit_body(k, ans):
        shift = jnp.uint32(31) - k.astype(jnp.uint32)
        cand = ans | lax.shift_left(jnp.uint32(1), shift)
        return jnp.where(count_cmp(dec(cand), False) >= C, cand, ans)

    thr = dec(lax.fori_loop(0, 32, bit_body, jnp.uint32(0)))

    ties_take = C - count_cmp(thr, True)

    base = s * _VPT

    def pre_body(j, carry):
        g, e = carry
        v = scores_v[pl.ds(j * LANES, LANES)]
        g = g + jnp.where(v > thr, 1, 0).astype(jnp.int32)
        e = e + jnp.where(v == thr, 1, 0).astype(jnp.int32)
        return (g, e)

    zv = jnp.zeros((LANES,), jnp.int32)
    g_pre, e_pre = lax.fori_loop(0, s * (_VPT // LANES), pre_body, (zv, zv))
    gt_acc = _lanesum(g_pre)
    eq_acc = _lanesum(e_pre)

    lane = lax.iota(jnp.int32, LANES)
    for jj in range(_VPT // LANES):
        off = base + jj * LANES
        sv = scores_v[pl.ds(off, LANES)]
        gtb = sv > thr
        eqb = sv == thr
        gt = jnp.where(gtb, 1, 0).astype(jnp.int32)
        eq = jnp.where(eqb, 1, 0).astype(jnp.int32)
        exc_gt = lax.broadcast(gt_acc, (LANES,))
        exc_eq = lax.broadcast(eq_acc, (LANES,))
        for j in range(LANES - 1):
            later = lane > j
            exc_gt = exc_gt + jnp.where(later, gt[j], 0)
            exc_eq = exc_eq + jnp.where(later, eq[j], 0)
        sel = gtb | (eqb & (exc_eq < ties_take))
        sel_before = exc_gt + jnp.minimum(exc_eq, ties_take)
        nvec = off + lane
        pos = jnp.where(sel, sel_before, C + nvec - sel_before)
        posb[pl.ds(jj * LANES, LANES)] = pos
        valb[pl.ds(jj * LANES, LANES)] = nvec + b * N
        gateb[pl.ds(jj * LANES, LANES)] = 1.0 / (1.0 + jnp.exp(-sv))
        gt_acc = exc_gt[LANES - 1] + gt[LANES - 1]
        eq_acc = exc_eq[LANES - 1] + eq[LANES - 1]

    pltpu.sync_copy(valb, sel_sh.at[posb])
    pltpu.sync_copy(gateb, gate_sh.at[posb])
    plsc.subcore_barrier()

    slot0 = pl.multiple_of(s * _VPT, _VPT)
    pltpu.sync_copy(sel_sh.at[pl.ds(slot0, _VPT)], selv)
    pltpu.sync_copy(
        selv, selfull_hbm.at[pl.ds(pl.multiple_of(b * N + slot0, _VPT), _VPT)]
    )

    @pl.when(slot0 < C)
    def _():
        pltpu.sync_copy(gate_sh.at[pl.ds(slot0, _VPT)], gatev)
        pltpu.sync_copy(
            gatev,
            gate_hbm.at[pl.ds(pl.multiple_of(b * C + slot0, _VPT), _VPT)],
        )

    g0 = pl.multiple_of(s * _GPT, _GPT)
    pltpu.sync_copy(sel_sh.at[pl.ds(g0, _GPT)], gidx)
    for k in range(_GPT // LANES):
        pltpu.async_copy(
            x_hbm.at[gidx.at[pl.ds(k * LANES, LANES)]], rowb, sem
        ).wait()
        pltpu.sync_copy(
            rowb,
            xin_hbm.at[
                pl.ds(pl.multiple_of(b * C + g0 + k * LANES, LANES), LANES)
            ],
        )


def _select_gather_build():
    return pl.kernel(
        _select_gather_body,
        out_type=[
            jax.ShapeDtypeStruct((B * N,), jnp.int32),
            jax.ShapeDtypeStruct((B * C,), jnp.float32),
            jax.ShapeDtypeStruct((B * C, D), jnp.float32),
        ],
        mesh=_MESH,
        scratch_types=[
            pltpu.VMEM((N,), jnp.float32),
            pltpu.VMEM((_VPT,), jnp.int32),
            pltpu.VMEM((_VPT,), jnp.int32),
            pltpu.VMEM((_VPT,), jnp.float32),
            pltpu.VMEM((_VPT,), jnp.int32),
            pltpu.VMEM((_VPT,), jnp.float32),
            pltpu.VMEM((_GPT,), jnp.int32),
            pltpu.VMEM((LANES, D), jnp.float32),
            pltpu.SemaphoreType.DMA,
            pltpu.VMEM_SHARED((N,), jnp.int32),
            pltpu.VMEM_SHARED((N,), jnp.float32),
        ],
    )


def _dot1(a, bm, dims=None):
    if dims is None:
        dims = (((1,), (0,)), ((), ()))
    return lax.dot_general(
        a, bm, dimension_numbers=dims, preferred_element_type=jnp.float32
    )


def _qkv_body(x_ref, wq_ref, wk_ref, wv_ref, q_ref, k_ref, v_ref):
    x = x_ref[...]
    q_ref[...] = _dot1(x, wq_ref[...])
    k_ref[...] = _dot1(x, wk_ref[...])
    v_ref[...] = _dot1(x, wv_ref[...])


def _qkv(xin, Wq, Wk, Wv):
    rblk, cblk = 512, 512
    ni, nj = (B * C) // rblk, D // cblk
    shp = jax.ShapeDtypeStruct((B * C, D), jnp.float32)
    return pl.pallas_call(
        _qkv_body,
        grid=(ni, nj),
        in_specs=[
            pl.BlockSpec((rblk, D), lambda i, j: (i, 0)),
            pl.BlockSpec((D, cblk), lambda i, j: (0, j)),
            pl.BlockSpec((D, cblk), lambda i, j: (0, j)),
            pl.BlockSpec((D, cblk), lambda i, j: (0, j)),
        ],
        out_specs=[
            pl.BlockSpec((rblk, cblk), lambda i, j: (i, j)),
            pl.BlockSpec((rblk, cblk), lambda i, j: (i, j)),
            pl.BlockSpec((rblk, cblk), lambda i, j: (i, j)),
        ],
        out_shape=[shp, shp, shp],
    )(xin, Wq, Wk, Wv)


def _head_body(q_ref, k_ref, v_ref, o_ref):
    logits = _dot1(
        q_ref[...], k_ref[...], dims=(((1,), (1,)), ((), ()))
    ) * (1.0 / math.sqrt(DH))
    m = jnp.max(logits, axis=-1, keepdims=True)
    p = jnp.exp(logits - m)
    p = p / jnp.sum(p, axis=-1, keepdims=True)
    o_ref[...] = _dot1(p, v_ref[...])


def _heads(q, k, v):
    spec = pl.BlockSpec((C, DH), lambda b, h: (b, h))
    return pl.pallas_call(
        _head_body,
        grid=(B, H),
        in_specs=[spec, spec, spec],
        out_specs=pl.BlockSpec((C, DH), lambda b, h: (b, h)),
        out_shape=jax.ShapeDtypeStruct((B * C, D), jnp.float32),
    )(q, k, v)


def _outproj_body(o_ref, wo_ref, gate_ref, y_ref):
    o_r = _dot1(o_ref[...], wo_ref[...]).astype(jnp.bfloat16)
    g_r = gate_ref[...].astype(jnp.bfloat16)
    y_ref[...] = o_r.astype(jnp.float32) * g_r.astype(jnp.float32)


def _outproj(o_heads, Wo, gate_col):
    rblk, cblk = 512, 512
    ni, nj = (B * C) // rblk, D // cblk
    return pl.pallas_call(
        _outproj_body,
        grid=(ni, nj),
        in_specs=[
            pl.BlockSpec((rblk, D), lambda i, j: (i, 0)),
            pl.BlockSpec((D, cblk), lambda i, j: (0, j)),
            pl.BlockSpec((rblk, 1), lambda i, j: (i, 0)),
        ],
        out_specs=pl.BlockSpec((rblk, cblk), lambda i, j: (i, j)),
        out_shape=jax.ShapeDtypeStruct((B * C, D), jnp.float32),
    )(o_heads, Wo, gate_col)


def _scatter_body(osc_hbm, sel2d_hbm, out_hbm, idxv, rowb, sem):
    b = lax.axis_index("c")
    s = lax.axis_index("s")
    slot0 = pl.multiple_of(s * _VPT, _VPT)
    row0 = pl.multiple_of((b * N + slot0) // LANES, _VPT // LANES)
    pltpu.sync_copy(sel2d_hbm.at[pl.ds(row0, _VPT // LANES)], idxv)
    nsel_tiles = C // _VPT

    @pl.when(s >= nsel_tiles)
    def _():
        def zb(i, carry):
            def zl(j, carry2):
                rowb[i, pl.ds(j * LANES, LANES)] = jnp.zeros(
                    (LANES,), jnp.float32
                )
                return carry2

            return lax.fori_loop(0, D // LANES, zl, carry)

        lax.fori_loop(0, LANES, zb, 0)

    for k in range(_VPT // LANES):
        @pl.when(s < nsel_tiles)
        def _():
            pltpu.sync_copy(
                osc_hbm.at[
                    pl.ds(
                        pl.multiple_of(b * C + slot0 + k * LANES, LANES),
                        LANES,
                    )
                ],
                rowb,
            )

        pltpu.async_copy(rowb, out_hbm.at[idxv.at[k]], sem).wait()


def _scatter_build():
    return pl.kernel(
        _scatter_body,
        out_type=jax.ShapeDtypeStruct((B * N, D), jnp.float32),
        mesh=_MESH,
        scratch_types=[
            pltpu.VMEM((_VPT // LANES, LANES), jnp.int32),
            pltpu.VMEM((LANES, D), jnp.float32),
            pltpu.SemaphoreType.DMA,
        ],
    )


def kernel(token_inputs, w_router, Wq, Wk, Wv, Wo):
    x2d = token_inputs.reshape(B * N, D)
    w_row = w_router.reshape(1, D)

    scores = _scores(x2d, w_row).reshape(B * N)
    selfull, gate, xin = _select_gather_build()(scores, x2d)
    gate_col = gate.reshape(B * C, 1)
    q, k, v = _qkv(xin, Wq, Wk, Wv)
    o_heads = _heads(q, k, v)
    osc = _outproj(o_heads, Wo, gate_col)
    sel2d = selfull.reshape(B * N // LANES, LANES)
    out2d = _scatter_build()(osc, sel2d)
    return out2d.reshape(B, N, D)


# speedup vs baseline: 1.0174x; 1.0174x over previous
import functools
import math

import jax
import jax.numpy as jnp
from jax import lax
from jax.experimental import pallas as pl
from jax.experimental.pallas import tpu as pltpu
from jax.experimental.pallas import tpu_sc as plsc

B, N, D, H = 2, 2048, 2048, 16
C = N // 2
DH = D // H
NC, NS, LANES = 2, 16, 16

_MESH = plsc.VectorSubcoreMesh(
    core_axis_name="c", subcore_axis_name="s", num_cores=NC, num_subcores=NS
)

_VPT = N // NS
_GPT = C // NS


def _scores_body(x_ref, w_ref, o_ref):
    xb = x_ref[...].astype(jnp.bfloat16).astype(jnp.float32)
    wb = w_ref[...].astype(jnp.bfloat16).astype(jnp.float32)
    o_ref[...] = jnp.sum(xb * wb, axis=1)[None, None, :]


def _scores(x2d, w_row):
    nblk = 8
    rows = (B * N) // nblk
    out = pl.pallas_call(
        _scores_body,
        grid=(nblk,),
        in_specs=[
            pl.BlockSpec((rows, D), lambda i: (i, 0)),
            pl.BlockSpec((1, D), lambda i: (0, 0)),
        ],
        out_specs=pl.BlockSpec((1, 1, rows), lambda i: (i, 0, 0)),
        out_shape=jax.ShapeDtypeStruct((nblk, 1, rows), jnp.float32),
    )(x2d, w_row)
    return out.reshape(B, N)


def _select_gather_body(
    scores_hbm, x_hbm,
    selfull_hbm, gate_hbm, xin_hbm,
    scores_v, posb, valb, gateb, selv, gatev, gidx, rowb, sem,
    sel_sh, gate_sh,
):
    b = lax.axis_index("c")
    s = lax.axis_index("s")

    pltpu.sync_copy(scores_hbm.at[pl.ds(pl.multiple_of(b * N, N), N)], scores_v)

    def dec(e):
        big = e >= jnp.uint32(0x80000000)
        bits = jnp.where(big, e & jnp.uint32(0x7FFFFFFF), ~e)
        return lax.bitcast_convert_type(bits, jnp.float32)

    def _lanesum(vec):
        tot = vec[0]
        for i in range(1, LANES):
            tot = tot + vec[i]
        return tot

    def count_cmp(thr, strict):
        def cb(j, acc):
            v = scores_v[pl.ds(j * LANES, LANES)]
            hit = (v > thr) if strict else (v >= thr)
            return acc + jnp.where(hit, 1, 0).astype(jnp.int32)

        acc = lax.fori_loop(
            0, N // LANES, cb, jnp.zeros((LANES,), jnp.int32)
        )
        return _lanesum(acc)

    def bit_body(k, ans):
        shift = jnp.uint32(31) - k.astype(jnp.uint32)
        cand = ans | lax.shift_left(jnp.uint32(1), shift)
        return jnp.where(count_cmp(dec(cand), False) >= C, cand, ans)

    thr = dec(lax.fori_loop(0, 32, bit_body, jnp.uint32(0)))

    ties_take = C - count_cmp(thr, True)

    base = s * _VPT

    def pre_body(j, carry):
        g, e = carry
        v = scores_v[pl.ds(j * LANES, LANES)]
        g = g + jnp.where(v > thr, 1, 0).astype(jnp.int32)
        e = e + jnp.where(v == thr, 1, 0).astype(jnp.int32)
        return (g, e)

    zv = jnp.zeros((LANES,), jnp.int32)
    g_pre, e_pre = lax.fori_loop(0, s * (_VPT // LANES), pre_body, (zv, zv))
    gt_acc = _lanesum(g_pre)
    eq_acc = _lanesum(e_pre)

    lane = lax.iota(jnp.int32, LANES)
    for jj in range(_VPT // LANES):
        off = base + jj * LANES
        sv = scores_v[pl.ds(off, LANES)]
        gtb = sv > thr
        eqb = sv == thr
        gt = jnp.where(gtb, 1, 0).astype(jnp.int32)
        eq = jnp.where(eqb, 1, 0).astype(jnp.int32)
        exc_gt = lax.broadcast(gt_acc, (LANES,))
        exc_eq = lax.broadcast(eq_acc, (LANES,))
        for j in range(LANES - 1):
            later = lane > j
            exc_gt = exc_gt + jnp.where(later, gt[j], 0)
            exc_eq = exc_eq + jnp.where(later, eq[j], 0)
        sel = gtb | (eqb & (exc_eq < ties_take))
        sel_before = exc_gt + jnp.minimum(exc_eq, ties_take)
        nvec = off + lane
        pos = jnp.where(sel, sel_before, C + nvec - sel_before)
        posb[pl.ds(jj * LANES, LANES)] = pos
        valb[pl.ds(jj * LANES, LANES)] = nvec + b * N
        gateb[pl.ds(jj * LANES, LANES)] = 1.0 / (1.0 + jnp.exp(-sv))
        gt_acc = exc_gt[LANES - 1] + gt[LANES - 1]
        eq_acc = exc_eq[LANES - 1] + eq[LANES - 1]

    pltpu.sync_copy(valb, sel_sh.at[posb])
    pltpu.sync_copy(gateb, gate_sh.at[posb])
    plsc.subcore_barrier()

    slot0 = pl.multiple_of(s * _VPT, _VPT)
    pltpu.sync_copy(sel_sh.at[pl.ds(slot0, _VPT)], selv)
    pltpu.sync_copy(
        selv, selfull_hbm.at[pl.ds(pl.multiple_of(b * N + slot0, _VPT), _VPT)]
    )

    @pl.when(slot0 < C)
    def _():
        pltpu.sync_copy(gate_sh.at[pl.ds(slot0, _VPT)], gatev)
        pltpu.sync_copy(
            gatev,
            gate_hbm.at[pl.ds(pl.multiple_of(b * C + slot0, _VPT), _VPT)],
        )

    g0 = pl.multiple_of(s * _GPT, _GPT)
    pltpu.sync_copy(sel_sh.at[pl.ds(g0, _GPT)], gidx)
    for k in range(_GPT // LANES):
        pltpu.async_copy(
            x_hbm.at[gidx.at[pl.ds(k * LANES, LANES)]], rowb, sem
        ).wait()
        pltpu.sync_copy(
            rowb,
            xin_hbm.at[
                pl.ds(pl.multiple_of(b * C + g0 + k * LANES, LANES), LANES)
            ],
        )


def _select_gather_build():
    return pl.kernel(
        _select_gather_body,
        out_type=[
            jax.ShapeDtypeStruct((B * N,), jnp.int32),
            jax.ShapeDtypeStruct((B * C,), jnp.float32),
            jax.ShapeDtypeStruct((B * C, D), jnp.float32),
        ],
        mesh=_MESH,
        scratch_types=[
            pltpu.VMEM((N,), jnp.float32),
            pltpu.VMEM((_VPT,), jnp.int32),
            pltpu.VMEM((_VPT,), jnp.int32),
            pltpu.VMEM((_VPT,), jnp.float32),
            pltpu.VMEM((_VPT,), jnp.int32),
            pltpu.VMEM((_VPT,), jnp.float32),
            pltpu.VMEM((_GPT,), jnp.int32),
            pltpu.VMEM((LANES, D), jnp.float32),
            pltpu.SemaphoreType.DMA,
            pltpu.VMEM_SHARED((N,), jnp.int32),
            pltpu.VMEM_SHARED((N,), jnp.float32),
        ],
    )


def _dot1(a, bm, dims=None):
    if dims is None:
        dims = (((1,), (0,)), ((), ()))
    return lax.dot_general(
        a, bm, dimension_numbers=dims, preferred_element_type=jnp.float32
    )


def _qkv_body(x_ref, wq_ref, wk_ref, wv_ref, q_ref, k_ref, v_ref):
    x = x_ref[...]
    q_ref[...] = _dot1(x, wq_ref[...]).astype(jnp.bfloat16)
    k_ref[...] = _dot1(x, wk_ref[...]).astype(jnp.bfloat16)
    v_ref[...] = _dot1(x, wv_ref[...]).astype(jnp.bfloat16)


def _qkv(xin, Wq, Wk, Wv):
    rblk, cblk = 512, 512
    ni, nj = (B * C) // rblk, D // cblk
    shp = jax.ShapeDtypeStruct((B * C, D), jnp.bfloat16)
    return pl.pallas_call(
        _qkv_body,
        grid=(nj, ni),
        in_specs=[
            pl.BlockSpec((rblk, D), lambda j, i: (i, 0)),
            pl.BlockSpec((D, cblk), lambda j, i: (0, j)),
            pl.BlockSpec((D, cblk), lambda j, i: (0, j)),
            pl.BlockSpec((D, cblk), lambda j, i: (0, j)),
        ],
        out_specs=[
            pl.BlockSpec((rblk, cblk), lambda j, i: (i, j)),
            pl.BlockSpec((rblk, cblk), lambda j, i: (i, j)),
            pl.BlockSpec((rblk, cblk), lambda j, i: (i, j)),
        ],
        out_shape=[shp, shp, shp],
    )(xin, Wq, Wk, Wv)


def _attn_body(q_ref, k_ref, v_ref, wo_ref, gate_ref, o_ref):
    h = pl.program_id(1)
    logits = _dot1(
        q_ref[...], k_ref[...], dims=(((1,), (1,)), ((), ()))
    ) * (1.0 / math.sqrt(DH))
    p = jnp.exp(logits)
    inv = 1.0 / jnp.sum(p, axis=-1, keepdims=True)
    p = p * inv
    o_h = _dot1(p, v_ref[...])
    contrib = _dot1(o_h, wo_ref[...])

    @pl.when(h == 0)
    def _():
        o_ref[...] = contrib

    @pl.when(h > 0)
    def _():
        o_ref[...] = o_ref[...] + contrib

    @pl.when(h == H - 1)
    def _():
        o_r = o_ref[...].astype(jnp.bfloat16).astype(jnp.float32)
        g_r = gate_ref[...].astype(jnp.bfloat16).astype(jnp.float32)
        o_ref[...] = o_r * g_r


def _attention(q, k, v, Wo, gate_col):
    spec = pl.BlockSpec((C, DH), lambda b, h: (b, h))
    return pl.pallas_call(
        _attn_body,
        grid=(B, H),
        in_specs=[
            spec, spec, spec,
            pl.BlockSpec((DH, D), lambda b, h: (h, 0)),
            pl.BlockSpec((C, 1), lambda b, h: (b, 0)),
        ],
        out_specs=pl.BlockSpec((C, D), lambda b, h: (b, 0)),
        out_shape=jax.ShapeDtypeStruct((B * C, D), jnp.float32),
        compiler_params=pltpu.CompilerParams(
            dimension_semantics=("arbitrary", "arbitrary"),
        ),
    )(q, k, v, Wo, gate_col)


def _scatter_body(osc_hbm, sel2d_hbm, out_hbm, idxv, rowb, sem):
    b = lax.axis_index("c")
    s = lax.axis_index("s")
    slot0 = pl.multiple_of(s * _VPT, _VPT)
    row0 = pl.multiple_of((b * N + slot0) // LANES, _VPT // LANES)
    pltpu.sync_copy(sel2d_hbm.at[pl.ds(row0, _VPT // LANES)], idxv)
    nsel_tiles = C // _VPT

    @pl.when(s >= nsel_tiles)
    def _():
        def zb(i, carry):
            def zl(j, carry2):
                rowb[i, pl.ds(j * LANES, LANES)] = jnp.zeros(
                    (LANES,), jnp.float32
                )
                return carry2

            return lax.fori_loop(0, D // LANES, zl, carry)

        lax.fori_loop(0, LANES, zb, 0)

    for k in range(_VPT // LANES):
        @pl.when(s < nsel_tiles)
        def _():
            pltpu.sync_copy(
                osc_hbm.at[
                    pl.ds(
                        pl.multiple_of(b * C + slot0 + k * LANES, LANES),
                        LANES,
                    )
                ],
                rowb,
            )

        pltpu.async_copy(rowb, out_hbm.at[idxv.at[k]], sem).wait()


def _scatter_build():
    return pl.kernel(
        _scatter_body,
        out_type=jax.ShapeDtypeStruct((B * N, D), jnp.float32),
        mesh=_MESH,
        scratch_types=[
            pltpu.VMEM((_VPT // LANES, LANES), jnp.int32),
            pltpu.VMEM((LANES, D), jnp.float32),
            pltpu.SemaphoreType.DMA,
        ],
    )


def kernel(token_inputs, w_router, Wq, Wk, Wv, Wo):
    x2d = token_inputs.reshape(B * N, D)
    w_row = w_router.reshape(1, D)

    scores = _scores(x2d, w_row).reshape(B * N)
    selfull, gate, xin = _select_gather_build()(scores, x2d)
    gate_col = gate.reshape(B * C, 1)
    q, k, v = _qkv(xin, Wq, Wk, Wv)
    osc = _attention(q, k, v, Wo, gate_col)
    sel2d = selfull.reshape(B * N // LANES, LANES)
    out2d = _scatter_build()(osc, sel2d)
    return out2d.reshape(B, N, D)


# speedup vs baseline: 1.0694x; 1.0511x over previous
import functools
import math

import jax
import jax.numpy as jnp
from jax import lax
from jax.experimental import pallas as pl
from jax.experimental.pallas import tpu as pltpu
from jax.experimental.pallas import tpu_sc as plsc

B, N, D, H = 2, 2048, 2048, 16
C = N // 2
DH = D // H
NC, NS, LANES = 2, 16, 16

_MESH = plsc.VectorSubcoreMesh(
    core_axis_name="c", subcore_axis_name="s", num_cores=NC, num_subcores=NS
)

_VPT = N // NS
_GPT = C // NS


def _scores_body(x_ref, w_ref, o_ref):
    xb = x_ref[...].astype(jnp.bfloat16).astype(jnp.float32)
    wb = w_ref[...].astype(jnp.bfloat16).astype(jnp.float32)
    o_ref[...] = jnp.sum(xb * wb, axis=1)[None, None, :]


def _scores(x2d, w_row):
    nblk = 8
    rows = (B * N) // nblk
    out = pl.pallas_call(
        _scores_body,
        grid=(nblk,),
        in_specs=[
            pl.BlockSpec((rows, D), lambda i: (i, 0)),
            pl.BlockSpec((1, D), lambda i: (0, 0)),
        ],
        out_specs=pl.BlockSpec((1, 1, rows), lambda i: (i, 0, 0)),
        out_shape=jax.ShapeDtypeStruct((nblk, 1, rows), jnp.float32),
    )(x2d, w_row)
    return out.reshape(B, N)


def _select_gather_body(
    sbits_hbm, scores_hbm, x_hbm,
    selfull_hbm, gate_hbm, xin_hbm,
    bits_v, u_v, scores_v, posb, valb, gateb, selv, gatev, gidx,
    rowb0, rowb1, sem0, sem1,
    sel_sh, gate_sh,
):
    b = lax.axis_index("c")
    s = lax.axis_index("s")

    pltpu.sync_copy(sbits_hbm.at[pl.ds(pl.multiple_of(b * N, N), N)], bits_v)
    pltpu.sync_copy(scores_hbm.at[pl.ds(pl.multiple_of(b * N, N), N)], scores_v)

    def enc_body(j, carry):
        w = bits_v[pl.ds(j * LANES, LANES)]
        u = w.astype(jnp.uint32)
        u_v[pl.ds(j * LANES, LANES)] = jnp.where(
            u >= jnp.uint32(0x80000000), ~u, u | jnp.uint32(0x80000000)
        )
        return carry

    lax.fori_loop(0, N // LANES, enc_body, 0, unroll=4)

    def _lanesum(vec):
        tot = vec[0]
        for i in range(1, LANES):
            tot = tot + vec[i]
        return tot

    def count_ge(thr):
        def cb(j, acc):
            u = u_v[pl.ds(j * LANES, LANES)]
            return acc + jnp.where(u >= thr, 1, 0).astype(jnp.int32)

        acc = lax.fori_loop(
            0, N // LANES, cb, jnp.zeros((LANES,), jnp.int32), unroll=8
        )
        return _lanesum(acc)

    def bit_body(kk, a):
        shift = jnp.uint32(31) - kk.astype(jnp.uint32)
        cand = a | lax.shift_left(jnp.uint32(1), shift)
        return jnp.where(count_ge(cand) >= C, cand, a)

    ans = lax.fori_loop(0, 32, bit_body, jnp.uint32(0))
    ties_take = C - count_ge(ans + jnp.uint32(1))

    base = s * _VPT

    def pre_body(j, carry):
        g, e = carry
        u = u_v[pl.ds(j * LANES, LANES)]
        g = g + jnp.where(u > ans, 1, 0).astype(jnp.int32)
        e = e + jnp.where(u == ans, 1, 0).astype(jnp.int32)
        return (g, e)

    zv = jnp.zeros((LANES,), jnp.int32)
    g_pre, e_pre = lax.fori_loop(0, s * (_VPT // LANES), pre_body, (zv, zv))
    gt_acc = _lanesum(g_pre)
    eq_acc = _lanesum(e_pre)

    lane = lax.iota(jnp.int32, LANES)
    for jj in range(_VPT // LANES):
        off = base + jj * LANES
        sv = scores_v[pl.ds(off, LANES)]
        u = u_v[pl.ds(off, LANES)]
        gtb = u > ans
        eqb = u == ans
        gt = jnp.where(gtb, 1, 0).astype(jnp.int32)
        eq = jnp.where(eqb, 1, 0).astype(jnp.int32)
        exc_gt = lax.broadcast(gt_acc, (LANES,))
        exc_eq = lax.broadcast(eq_acc, (LANES,))
        for j in range(LANES - 1):
            later = lane > j
            exc_gt = exc_gt + jnp.where(later, gt[j], 0)
            exc_eq = exc_eq + jnp.where(later, eq[j], 0)
        sel = gtb | (eqb & (exc_eq < ties_take))
        sel_before = exc_gt + jnp.minimum(exc_eq, ties_take)
        nvec = off + lane
        pos = jnp.where(sel, sel_before, C + nvec - sel_before)
        posb[pl.ds(jj * LANES, LANES)] = pos
        valb[pl.ds(jj * LANES, LANES)] = nvec + b * N
        gateb[pl.ds(jj * LANES, LANES)] = 1.0 / (1.0 + jnp.exp(-sv))
        gt_acc = exc_gt[LANES - 1] + gt[LANES - 1]
        eq_acc = exc_eq[LANES - 1] + eq[LANES - 1]

    pltpu.sync_copy(valb, sel_sh.at[posb])
    pltpu.sync_copy(gateb, gate_sh.at[posb])
    plsc.subcore_barrier()

    slot0 = pl.multiple_of(s * _VPT, _VPT)
    pltpu.sync_copy(sel_sh.at[pl.ds(slot0, _VPT)], selv)
    pltpu.sync_copy(
        selv, selfull_hbm.at[pl.ds(pl.multiple_of(b * N + slot0, _VPT), _VPT)]
    )

    @pl.when(slot0 < C)
    def _():
        pltpu.sync_copy(gate_sh.at[pl.ds(slot0, _VPT)], gatev)
        pltpu.sync_copy(
            gatev,
            gate_hbm.at[pl.ds(pl.multiple_of(b * C + slot0, _VPT), _VPT)],
        )

    g0 = pl.multiple_of(s * _GPT, _GPT)
    pltpu.sync_copy(sel_sh.at[pl.ds(g0, _GPT)], gidx)
    bufs = (rowb0, rowb1)
    sems = (sem0, sem1)
    nchunk = _GPT // LANES

    def _start(k):
        return pltpu.async_copy(
            x_hbm.at[gidx.at[pl.ds(k * LANES, LANES)]],
            bufs[k % 2], sems[k % 2],
        )

    descs = [None] * nchunk
    descs[0] = _start(0)
    for k in range(nchunk):
        descs[k].wait()
        if k + 1 < nchunk:
            descs[k + 1] = _start(k + 1)
        pltpu.sync_copy(
            bufs[k % 2],
            xin_hbm.at[
                pl.ds(pl.multiple_of(b * C + g0 + k * LANES, LANES), LANES)
            ],
        )


def _select_gather_build():
    return pl.kernel(
        _select_gather_body,
        out_type=[
            jax.ShapeDtypeStruct((B * N,), jnp.int32),
            jax.ShapeDtypeStruct((B * C,), jnp.float32),
            jax.ShapeDtypeStruct((B * C, D), jnp.float32),
        ],
        mesh=_MESH,
        scratch_types=[
            pltpu.VMEM((N,), jnp.int32),
            pltpu.VMEM((N,), jnp.uint32),
            pltpu.VMEM((N,), jnp.float32),
            pltpu.VMEM((_VPT,), jnp.int32),
            pltpu.VMEM((_VPT,), jnp.int32),
            pltpu.VMEM((_VPT,), jnp.float32),
            pltpu.VMEM((_VPT,), jnp.int32),
            pltpu.VMEM((_VPT,), jnp.float32),
            pltpu.VMEM((_GPT,), jnp.int32),
            pltpu.VMEM((LANES, D), jnp.float32),
            pltpu.VMEM((LANES, D), jnp.float32),
            pltpu.SemaphoreType.DMA,
            pltpu.SemaphoreType.DMA,
            pltpu.VMEM_SHARED((N,), jnp.int32),
            pltpu.VMEM_SHARED((N,), jnp.float32),
        ],
    )


def _dot1(a, bm, dims=None):
    if dims is None:
        dims = (((1,), (0,)), ((), ()))
    return lax.dot_general(
        a, bm, dimension_numbers=dims, preferred_element_type=jnp.float32
    )


def _qkv_body(x_ref, wq_ref, wk_ref, wv_ref, q_ref, k_ref, v_ref):
    x = x_ref[...]
    q_ref[...] = _dot1(x, wq_ref[...]).astype(jnp.bfloat16)
    k_ref[...] = _dot1(x, wk_ref[...]).astype(jnp.bfloat16)
    v_ref[...] = _dot1(x, wv_ref[...]).astype(jnp.bfloat16)


def _qkv(xin, Wq, Wk, Wv):
    rblk, cblk = 512, 512
    ni, nj = (B * C) // rblk, D // cblk
    shp = jax.ShapeDtypeStruct((B * C, D), jnp.bfloat16)
    return pl.pallas_call(
        _qkv_body,
        grid=(nj, ni),
        in_specs=[
            pl.BlockSpec((rblk, D), lambda j, i: (i, 0)),
            pl.BlockSpec((D, cblk), lambda j, i: (0, j)),
            pl.BlockSpec((D, cblk), lambda j, i: (0, j)),
            pl.BlockSpec((D, cblk), lambda j, i: (0, j)),
        ],
        out_specs=[
            pl.BlockSpec((rblk, cblk), lambda j, i: (i, j)),
            pl.BlockSpec((rblk, cblk), lambda j, i: (i, j)),
            pl.BlockSpec((rblk, cblk), lambda j, i: (i, j)),
        ],
        out_shape=[shp, shp, shp],
    )(xin, Wq, Wk, Wv)


def _attn_body(q_ref, k_ref, v_ref, wo_ref, gate_ref, o_ref):
    h = pl.program_id(1)
    logits = _dot1(
        q_ref[...], k_ref[...], dims=(((1,), (1,)), ((), ()))
    ) * (1.0 / math.sqrt(DH))
    p = jnp.exp(logits)
    inv = 1.0 / jnp.sum(p, axis=-1, keepdims=True)
    p = p * inv
    o_h = _dot1(p, v_ref[...])
    contrib = _dot1(o_h, wo_ref[...])

    @pl.when(h == 0)
    def _():
        o_ref[...] = contrib

    @pl.when(h > 0)
    def _():
        o_ref[...] = o_ref[...] + contrib

    @pl.when(h == H - 1)
    def _():
        o_r = o_ref[...].astype(jnp.bfloat16).astype(jnp.float32)
        g_r = gate_ref[...].astype(jnp.bfloat16).astype(jnp.float32)
        o_ref[...] = o_r * g_r


def _attention(q, k, v, Wo, gate_col):
    spec = pl.BlockSpec((C, DH), lambda b, h: (b, h))
    return pl.pallas_call(
        _attn_body,
        grid=(B, H),
        in_specs=[
            spec, spec, spec,
            pl.BlockSpec((DH, D), lambda b, h: (h, 0)),
            pl.BlockSpec((C, 1), lambda b, h: (b, 0)),
        ],
        out_specs=pl.BlockSpec((C, D), lambda b, h: (b, 0)),
        out_shape=jax.ShapeDtypeStruct((B * C, D), jnp.float32),
        compiler_params=pltpu.CompilerParams(
            dimension_semantics=("arbitrary", "arbitrary"),
        ),
    )(q, k, v, Wo, gate_col)


def _scatter_body(osc_hbm, sel2d_hbm, out_hbm, idxv, rowb0, rowb1, sem0, sem1):
    b = lax.axis_index("c")
    s = lax.axis_index("s")
    slot0 = pl.multiple_of(s * _VPT, _VPT)
    row0 = pl.multiple_of((b * N + slot0) // LANES, _VPT // LANES)
    pltpu.sync_copy(sel2d_hbm.at[pl.ds(row0, _VPT // LANES)], idxv)
    nsel_tiles = C // _VPT
    bufs = (rowb0, rowb1)
    sems = (sem0, sem1)

    @pl.when(s >= nsel_tiles)
    def _():
        def zb(i, carry):
            def zl(j, carry2):
                rowb0[i, pl.ds(j * LANES, LANES)] = jnp.zeros(
                    (LANES,), jnp.float32
                )
                rowb1[i, pl.ds(j * LANES, LANES)] = jnp.zeros(
                    (LANES,), jnp.float32
                )
                return carry2

            return lax.fori_loop(0, D // LANES, zl, carry)

        lax.fori_loop(0, LANES, zb, 0)

    descs = [None, None]
    for k in range(_VPT // LANES):
        if descs[k % 2] is not None:
            descs[k % 2].wait()

        @pl.when(s < nsel_tiles)
        def _(k=k):
            pltpu.sync_copy(
                osc_hbm.at[
                    pl.ds(
                        pl.multiple_of(b * C + slot0 + k * LANES, LANES),
                        LANES,
                    )
                ],
                bufs[k % 2],
            )

        descs[k % 2] = pltpu.async_copy(
            bufs[k % 2], out_hbm.at[idxv.at[k]], sems[k % 2]
        )
    descs[0].wait()
    descs[1].wait()


def _scatter_build():
    return pl.kernel(
        _scatter_body,
        out_type=jax.ShapeDtypeStruct((B * N, D), jnp.float32),
        mesh=_MESH,
        scratch_types=[
            pltpu.VMEM((_VPT // LANES, LANES), jnp.int32),
            pltpu.VMEM((LANES, D), jnp.float32),
            pltpu.VMEM((LANES, D), jnp.float32),
            pltpu.SemaphoreType.DMA,
            pltpu.SemaphoreType.DMA,
        ],
    )


def kernel(token_inputs, w_router, Wq, Wk, Wv, Wo):
    x2d = token_inputs.reshape(B * N, D)
    w_row = w_router.reshape(1, D)

    scores = _scores(x2d, w_row).reshape(B * N)
    sbits = lax.bitcast_convert_type(scores, jnp.int32)
    selfull, gate, xin = _select_gather_build()(sbits, scores, x2d)
    gate_col = gate.reshape(B * C, 1)
    q, k, v = _qkv(xin, Wq, Wk, Wv)
    osc = _attention(q, k, v, Wo, gate_col)
    sel2d = selfull.reshape(B * N // LANES, LANES)
    out2d = _scatter_build()(osc, sel2d)
    return out2d.reshape(B, N, D)
